# baseline (speedup 1.0000x reference)
import jax
import jax.numpy as jnp
from jax import lax
from jax.experimental import pallas as pl
from jax.experimental.pallas import tpu as pltpu
from jax.experimental.pallas import tpu_sc as plsc

VOCAB = 100000
EMBED_DIM = 128
N_CTX = 16
B = 1024
L = 200

_NC = 2
_NS = 16
_NW = _NC * _NS
_BPW = B // _NW
_NBUF = 4

_G0_OFF, _G0_LEN = 16, 96
_G1_OFF, _G1_LEN = 112, 88


def _sc_embed(ids_hbm, prefix_hbm, pos_hbm, posids_hbm, tok_hbm, out_hbm,
              idx_v, pidx_v, w_all, tsems, psems, osems):
  sid = lax.axis_index("s")
  wid = sid * _NC + lax.axis_index("c")
  base_b = wid * _BPW

  pltpu.sync_copy(ids_hbm.at[pl.ds(base_b * L, _BPW * L)], idx_v)
  pltpu.sync_copy(posids_hbm, pidx_v)

  for s in range(_NBUF):
    pltpu.sync_copy(prefix_hbm, w_all.at[s, pl.ds(0, N_CTX)])

  def start_tok(j, s):
    g0 = pltpu.async_copy(
        tok_hbm.at[idx_v.at[pl.ds(j * L + _G0_OFF, _G0_LEN)]],
        w_all.at[s, pl.ds(_G0_OFF, _G0_LEN)], tsems.at[s])
    g1 = pltpu.async_copy(
        tok_hbm.at[idx_v.at[pl.ds(j * L + _G1_OFF, _G1_LEN)]],
        w_all.at[s, pl.ds(_G1_OFF, _G1_LEN)], tsems.at[s])
    return g0, g1

  def start_pos(s):
    g0 = pltpu.async_copy(
        pos_hbm.at[pidx_v.at[pl.ds(_G0_OFF, _G0_LEN)]],
        w_all.at[s, pl.ds(_G0_OFF, _G0_LEN)], psems.at[s], add=True)
    g1 = pltpu.async_copy(
        pos_hbm.at[pidx_v.at[pl.ds(_G1_OFF, _G1_LEN)]],
        w_all.at[s, pl.ds(_G1_OFF, _G1_LEN)], psems.at[s], add=True)
    return g0, g1

  def start_out(j, s):
    return pltpu.async_copy(w_all.at[s], out_hbm.at[base_b + j],
                            osems.at[s])

  def wait_pair(sems, s):
    pltpu.make_async_copy(
        tok_hbm.at[pl.ds(0, _G0_LEN)],
        w_all.at[0, pl.ds(_G0_OFF, _G0_LEN)], sems.at[s]).wait()
    pltpu.make_async_copy(
        tok_hbm.at[pl.ds(0, _G1_LEN)],
        w_all.at[0, pl.ds(_G1_OFF, _G1_LEN)], sems.at[s]).wait()

  def wait_out(s):
    pltpu.make_async_copy(w_all.at[0], out_hbm.at[0], osems.at[s]).wait()

  for t in range(_NBUF):
    start_tok(t, t % _NBUF)
    if t >= 1:
      wait_pair(tsems, (t - 1) % _NBUF)
      start_pos((t - 1) % _NBUF)
    if t >= 2:
      wait_pair(psems, (t - 2) % _NBUF)
      start_out(t - 2, (t - 2) % _NBUF)

  def body(t, carry):
    s = lax.rem(t, _NBUF)
    wait_out(s)
    start_tok(t, s)
    s1 = lax.rem(t - 1, _NBUF)
    wait_pair(tsems, s1)
    start_pos(s1)
    s2 = lax.rem(t - 2, _NBUF)
    wait_pair(psems, s2)
    start_out(t - 2, s2)
    return carry

  lax.fori_loop(_NBUF, _BPW, body, 0)

  for t in range(_BPW, _BPW + 2):
    j = t - 1
    if j < _BPW:
      wait_pair(tsems, j % _NBUF)
      start_pos(j % _NBUF)
    j2 = t - 2
    wait_pair(psems, j2 % _NBUF)
    start_out(j2, j2 % _NBUF)
  for j in range(_BPW - _NBUF, _BPW):
    wait_out(j % _NBUF)


@jax.jit
def kernel(input_ids, token_table, position_table, ctx):
  ids = input_ids.astype(jnp.int32).reshape(-1)
  prefix = ctx[:N_CTX] + position_table[:N_CTX]
  pos_ids = jnp.arange(L, dtype=jnp.int32)

  mesh = plsc.VectorSubcoreMesh(core_axis_name="c", subcore_axis_name="s")
  run = pl.kernel(
      _sc_embed,
      out_type=jax.ShapeDtypeStruct((B, L, EMBED_DIM), jnp.float32),
      mesh=mesh,
      scratch_types=[
          pltpu.VMEM((_BPW * L,), jnp.int32),
          pltpu.VMEM((L,), jnp.int32),
          pltpu.VMEM((_NBUF, L, EMBED_DIM), jnp.float32),
          pltpu.SemaphoreType.DMA((_NBUF,)),
          pltpu.SemaphoreType.DMA((_NBUF,)),
          pltpu.SemaphoreType.DMA((_NBUF,)),
      ],
  )
  return run(ids, prefix, position_table, pos_ids, token_table)


# speedup vs baseline: 2.2294x; 2.2294x over previous
import jax
import jax.numpy as jnp
from jax import lax
from jax.experimental import pallas as pl
from jax.experimental.pallas import tpu as pltpu
from jax.experimental.pallas import tpu_sc as plsc

VOCAB = 100000
EMBED_DIM = 128
N_CTX = 16
B = 1024
L = 200

_NC = 2
_NS = 16
_NW = _NC * _NS
_BPW = B // _NW
_NBUF = 4
_NLANE = 16

_BT_OFF, _BT_LEN = N_CTX, 64
_AT_OFF, _AT_LEN = 80, 120


def _sc_embed(ids_hbm, base_hbm, tok_hbm, out_hbm,
              idx_v, w_all, tmpl_v, base_sh, isems, bsems, asems, osems):
  sid = lax.axis_index("s")
  wid = sid * _NC + lax.axis_index("c")
  base_b = wid * _BPW
  my_base = base_sh

  @pl.when(sid == 0)
  def _():
    pltpu.sync_copy(base_hbm, my_base)

  pltpu.sync_copy(ids_hbm.at[pl.ds(base_b * L, _BPW * L)], idx_v)
  pltpu.sync_copy(base_hbm.at[pl.ds(_BT_OFF, _BT_LEN)], tmpl_v)
  plsc.subcore_barrier()

  for s in range(_NBUF):
    pltpu.sync_copy(my_base.at[pl.ds(0, N_CTX)],
                    w_all.at[s, pl.ds(0, N_CTX)])

  def start_init(s):
    return pltpu.async_copy(
        my_base.at[pl.ds(_AT_OFF, _AT_LEN)],
        w_all.at[s, pl.ds(_AT_OFF, _AT_LEN)], isems.at[s])

  def start_tok_b(j, s):
    return pltpu.async_copy(
        tok_hbm.at[idx_v.at[pl.ds(j * L + _BT_OFF, _BT_LEN)]],
        w_all.at[s, pl.ds(_BT_OFF, _BT_LEN)], bsems.at[s])

  def start_tok_a(j, s):
    return pltpu.async_copy(
        tok_hbm.at[idx_v.at[pl.ds(j * L + _AT_OFF, _AT_LEN)]],
        w_all.at[s, pl.ds(_AT_OFF, _AT_LEN)], asems.at[s], add=True)

  def start_out(j, s):
    return pltpu.async_copy(w_all.at[s], out_hbm.at[base_b + j],
                            osems.at[s])

  def tec_add(s):
    def row_body(r, carry):
      row = _BT_OFF + r
      for c in range(EMBED_DIM // _NLANE):
        sl = pl.ds(c * _NLANE, _NLANE)
        w_all[s, row, sl] = w_all[s, row, sl] + tmpl_v[r, sl]
      return carry
    lax.fori_loop(0, _BT_LEN, row_body, 0)

  def wait_init(s):
    pltpu.make_async_copy(
        my_base.at[pl.ds(_AT_OFF, _AT_LEN)],
        w_all.at[0, pl.ds(_AT_OFF, _AT_LEN)], isems.at[s]).wait()

  def wait_tok_b(s):
    pltpu.make_async_copy(
        tok_hbm.at[pl.ds(0, _BT_LEN)],
        w_all.at[0, pl.ds(_BT_OFF, _BT_LEN)], bsems.at[s]).wait()

  def wait_tok_a(s):
    pltpu.make_async_copy(
        tok_hbm.at[pl.ds(0, _AT_LEN)],
        w_all.at[0, pl.ds(_AT_OFF, _AT_LEN)], asems.at[s]).wait()

  def wait_out(s):
    pltpu.make_async_copy(w_all.at[0], out_hbm.at[0], osems.at[s]).wait()

  for t in range(_NBUF):
    s = t % _NBUF
    start_init(s)
    start_tok_b(t, s)
    if t >= 1:
      s1 = (t - 1) % _NBUF
      wait_init(s1)
      start_tok_a(t - 1, s1)
      wait_tok_b(s1)
      tec_add(s1)
    if t >= 2:
      s2 = (t - 2) % _NBUF
      wait_tok_a(s2)
      start_out(t - 2, s2)

  def body(t, carry):
    s = lax.rem(t, _NBUF)
    wait_out(s)
    start_init(s)
    start_tok_b(t, s)
    s1 = lax.rem(t - 1, _NBUF)
    wait_init(s1)
    start_tok_a(t - 1, s1)
    wait_tok_b(s1)
    tec_add(s1)
    s2 = lax.rem(t - 2, _NBUF)
    wait_tok_a(s2)
    start_out(t - 2, s2)
    return carry

  lax.fori_loop(_NBUF, _BPW, body, 0)

  for t in range(_BPW, _BPW + 2):
    j = t - 1
    if j < _BPW:
      s1 = j % _NBUF
      wait_init(s1)
      start_tok_a(j, s1)
      wait_tok_b(s1)
      tec_add(s1)
    j2 = t - 2
    s2 = j2 % _NBUF
    wait_tok_a(s2)
    start_out(j2, s2)
  for j in range(_BPW - _NBUF, _BPW):
    wait_out(j % _NBUF)


@jax.jit
def kernel(input_ids, token_table, position_table, ctx):
  ids = input_ids.astype(jnp.int32).reshape(-1)
  prefix = ctx[:N_CTX] + position_table[:N_CTX]
  base = jnp.concatenate([prefix, position_table[N_CTX:L]], axis=0)

  mesh = plsc.VectorSubcoreMesh(core_axis_name="c", subcore_axis_name="s")
  run = pl.kernel(
      _sc_embed,
      out_type=jax.ShapeDtypeStruct((B, L, EMBED_DIM), jnp.float32),
      mesh=mesh,
      scratch_types=[
          pltpu.VMEM((_BPW * L,), jnp.int32),
          pltpu.VMEM((_NBUF, L, EMBED_DIM), jnp.float32),
          pltpu.VMEM((_BT_LEN, EMBED_DIM), jnp.float32),
          pltpu.VMEM_SHARED((L, EMBED_DIM), jnp.float32),
          pltpu.SemaphoreType.DMA((_NBUF,)),
          pltpu.SemaphoreType.DMA((_NBUF,)),
          pltpu.SemaphoreType.DMA((_NBUF,)),
          pltpu.SemaphoreType.DMA((_NBUF,)),
      ],
  )
  return run(ids, base, token_table)


# speedup vs baseline: 2.5567x; 1.1468x over previous
import jax
import jax.numpy as jnp
from jax import lax
from jax.experimental import pallas as pl
from jax.experimental.pallas import tpu as pltpu
from jax.experimental.pallas import tpu_sc as plsc

VOCAB = 100000
EMBED_DIM = 128
N_CTX = 16
B = 1024
L = 200

_NC = 2
_NS = 16
_NW = _NC * _NS
_BPW = B // _NW
_NBUF = 4
_NLANE = 16

_BT_OFF, _BT_LEN = N_CTX, 32
_AT_OFF, _AT_LEN = 48, 152
_A0_OFF, _A0_LEN = 48, 128
_A1_OFF, _A1_LEN = 176, 24


def _sc_embed(ids_hbm, base_hbm, tok_hbm, out_hbm,
              idx_v, w_all, tmpl_v, base_sh, isems, bsems, asems, osems):
  sid = lax.axis_index("s")
  wid = sid * _NC + lax.axis_index("c")
  base_b = wid * _BPW
  my_base = base_sh

  @pl.when(sid == 0)
  def _():
    pltpu.sync_copy(base_hbm, my_base)

  pltpu.sync_copy(ids_hbm.at[pl.ds(base_b * L, _BPW * L)], idx_v)
  pltpu.sync_copy(base_hbm.at[pl.ds(_BT_OFF, _BT_LEN)], tmpl_v)
  plsc.subcore_barrier()

  for s in range(_NBUF):
    pltpu.sync_copy(my_base.at[pl.ds(0, N_CTX)],
                    w_all.at[s, pl.ds(0, N_CTX)])

  def start_init(s):
    return pltpu.async_copy(
        my_base.at[pl.ds(_AT_OFF, _AT_LEN)],
        w_all.at[s, pl.ds(_AT_OFF, _AT_LEN)], isems.at[s])

  def start_tok_b(j, s):
    return pltpu.async_copy(
        tok_hbm.at[idx_v.at[pl.ds(j * L + _BT_OFF, _BT_LEN)]],
        w_all.at[s, pl.ds(_BT_OFF, _BT_LEN)], bsems.at[s])

  def start_tok_a(j, s):
    g0 = pltpu.async_copy(
        tok_hbm.at[idx_v.at[pl.ds(j * L + _A0_OFF, _A0_LEN)]],
        w_all.at[s, pl.ds(_A0_OFF, _A0_LEN)], asems.at[s], add=True)
    g1 = pltpu.async_copy(
        tok_hbm.at[idx_v.at[pl.ds(j * L + _A1_OFF, _A1_LEN)]],
        w_all.at[s, pl.ds(_A1_OFF, _A1_LEN)], asems.at[s], add=True)
    return g0, g1

  def start_out(j, s):
    return pltpu.async_copy(w_all.at[s], out_hbm.at[base_b + j],
                            osems.at[s])

  def tec_add(s):
    def row_body(r, carry):
      row = _BT_OFF + r
      for c in range(EMBED_DIM // _NLANE):
        sl = pl.ds(c * _NLANE, _NLANE)
        w_all[s, row, sl] = w_all[s, row, sl] + tmpl_v[r, sl]
      return carry
    lax.fori_loop(0, _BT_LEN, row_body, 0)

  def wait_init(s):
    pltpu.make_async_copy(
        my_base.at[pl.ds(_AT_OFF, _AT_LEN)],
        w_all.at[0, pl.ds(_AT_OFF, _AT_LEN)], isems.at[s]).wait()

  def wait_tok_b(s):
    pltpu.make_async_copy(
        tok_hbm.at[pl.ds(0, _BT_LEN)],
        w_all.at[0, pl.ds(_BT_OFF, _BT_LEN)], bsems.at[s]).wait()

  def wait_tok_a(s):
    pltpu.make_async_copy(
        tok_hbm.at[pl.ds(0, _A0_LEN)],
        w_all.at[0, pl.ds(_A0_OFF, _A0_LEN)], asems.at[s]).wait()
    pltpu.make_async_copy(
        tok_hbm.at[pl.ds(0, _A1_LEN)],
        w_all.at[0, pl.ds(_A1_OFF, _A1_LEN)], asems.at[s]).wait()

  def wait_out(s):
    pltpu.make_async_copy(w_all.at[0], out_hbm.at[0], osems.at[s]).wait()

  for t in range(_NBUF):
    s = t % _NBUF
    start_init(s)
    start_tok_b(t, s)
    if t >= 1:
      s1 = (t - 1) % _NBUF
      wait_init(s1)
      start_tok_a(t - 1, s1)
      wait_tok_b(s1)
      tec_add(s1)
    if t >= 2:
      s2 = (t - 2) % _NBUF
      wait_tok_a(s2)
      start_out(t - 2, s2)

  def body(t, carry):
    s = lax.rem(t, _NBUF)
    wait_out(s)
    start_init(s)
    start_tok_b(t, s)
    s1 = lax.rem(t - 1, _NBUF)
    wait_init(s1)
    start_tok_a(t - 1, s1)
    wait_tok_b(s1)
    tec_add(s1)
    s2 = lax.rem(t - 2, _NBUF)
    wait_tok_a(s2)
    start_out(t - 2, s2)
    return carry

  lax.fori_loop(_NBUF, _BPW, body, 0)

  for t in range(_BPW, _BPW + 2):
    j = t - 1
    if j < _BPW:
      s1 = j % _NBUF
      wait_init(s1)
      start_tok_a(j, s1)
      wait_tok_b(s1)
      tec_add(s1)
    j2 = t - 2
    s2 = j2 % _NBUF
    wait_tok_a(s2)
    start_out(j2, s2)
  for j in range(_BPW - _NBUF, _BPW):
    wait_out(j % _NBUF)


@jax.jit
def kernel(input_ids, token_table, position_table, ctx):
  ids = input_ids.astype(jnp.int32).reshape(-1)
  prefix = ctx[:N_CTX] + position_table[:N_CTX]
  base = jnp.concatenate([prefix, position_table[N_CTX:L]], axis=0)

  mesh = plsc.VectorSubcoreMesh(core_axis_name="c", subcore_axis_name="s")
  run = pl.kernel(
      _sc_embed,
      out_type=jax.ShapeDtypeStruct((B, L, EMBED_DIM), jnp.float32),
      mesh=mesh,
      scratch_types=[
          pltpu.VMEM((_BPW * L,), jnp.int32),
          pltpu.VMEM((_NBUF, L, EMBED_DIM), jnp.float32),
          pltpu.VMEM((_BT_LEN, EMBED_DIM), jnp.float32),
          pltpu.VMEM_SHARED((L, EMBED_DIM), jnp.float32),
          pltpu.SemaphoreType.DMA((_NBUF,)),
          pltpu.SemaphoreType.DMA((_NBUF,)),
          pltpu.SemaphoreType.DMA((_NBUF,)),
          pltpu.SemaphoreType.DMA((_NBUF,)),
      ],
  )
  return run(ids, base, token_table)


# speedup vs baseline: 2.5677x; 1.0043x over previous
import jax
import jax.numpy as jnp
from jax import lax
from jax.experimental import pallas as pl
from jax.experimental.pallas import tpu as pltpu
from jax.experimental.pallas import tpu_sc as plsc

VOCAB = 100000
EMBED_DIM = 128
N_CTX = 16
B = 1024
L = 200

_NC = 2
_NS = 16
_NW = _NC * _NS
_BPW = B // _NW

_G0_OFF, _G0_LEN = 16, 96
_G1_OFF, _G1_LEN = 112, 88


_NBUF = 4


def _sc_embed(ids_hbm, base_hbm, tok_hbm, out_hbm,
              idx_v, w_all, base_sh, isems, gsems, osems):
  sid = lax.axis_index("s")
  wid = sid * _NC + lax.axis_index("c")
  base_b = wid * _BPW
  my_base = base_sh

  @pl.when(sid == 0)
  def _():
    pltpu.sync_copy(base_hbm, my_base)

  pltpu.sync_copy(ids_hbm.at[pl.ds(base_b * L, _BPW * L)], idx_v)
  plsc.subcore_barrier()

  for s in range(_NBUF):
    pltpu.sync_copy(my_base.at[pl.ds(0, N_CTX)],
                    w_all.at[s, pl.ds(0, N_CTX)])

  def start_init(t, s):
    return pltpu.async_copy(
        my_base.at[pl.ds(N_CTX, L - N_CTX)],
        w_all.at[s, pl.ds(N_CTX, L - N_CTX)], isems.at[s])

  def start_gathers(j, s):
    g0 = pltpu.async_copy(
        tok_hbm.at[idx_v.at[pl.ds(j * L + _G0_OFF, _G0_LEN)]],
        w_all.at[s, pl.ds(_G0_OFF, _G0_LEN)], gsems.at[s], add=True)
    g1 = pltpu.async_copy(
        tok_hbm.at[idx_v.at[pl.ds(j * L + _G1_OFF, _G1_LEN)]],
        w_all.at[s, pl.ds(_G1_OFF, _G1_LEN)], gsems.at[s], add=True)
    return g0, g1

  def start_out(j, s):
    return pltpu.async_copy(w_all.at[s], out_hbm.at[base_b + j],
                            osems.at[s])

  def wait_init(s):
    pltpu.make_async_copy(
        my_base.at[pl.ds(N_CTX, L - N_CTX)],
        w_all.at[0, pl.ds(N_CTX, L - N_CTX)], isems.at[s]).wait()

  def wait_gathers(s):
    pltpu.make_async_copy(
        tok_hbm.at[pl.ds(0, _G0_LEN)],
        w_all.at[0, pl.ds(_G0_OFF, _G0_LEN)], gsems.at[s]).wait()
    pltpu.make_async_copy(
        tok_hbm.at[pl.ds(0, _G1_LEN)],
        w_all.at[0, pl.ds(_G1_OFF, _G1_LEN)], gsems.at[s]).wait()

  def wait_out(s):
    pltpu.make_async_copy(w_all.at[0], out_hbm.at[0], osems.at[s]).wait()

  init_d = {}
  gath_d = {}
  out_d = {}
  for t in range(_NBUF):
    init_d[t] = start_init(t, t % _NBUF)
    if t >= 1:
      init_d[t - 1].wait()
      gath_d[t - 1] = start_gathers(t - 1, (t - 1) % _NBUF)
    if t >= 2:
      gath_d[t - 2][0].wait()
      gath_d[t - 2][1].wait()
      out_d[t - 2] = start_out(t - 2, (t - 2) % _NBUF)

  def body(t, carry):
    s = lax.rem(t, _NBUF)
    wait_out(s)
    start_init(t, s)
    s1 = lax.rem(t - 1, _NBUF)
    wait_init(s1)
    start_gathers(t - 1, s1)
    s2 = lax.rem(t - 2, _NBUF)
    wait_gathers(s2)
    start_out(t - 2, s2)
    return carry

  lax.fori_loop(_NBUF, _BPW, body, 0)

  for t in range(_BPW, _BPW + 2):
    j = t - 1
    if j < _BPW:
      wait_init(j % _NBUF)
      gath_d[j] = start_gathers(j, j % _NBUF)
    j2 = t - 2
    wait_gathers(j2 % _NBUF)
    out_d[j2] = start_out(j2, j2 % _NBUF)
  for j in range(_BPW - _NBUF, _BPW):
    wait_out(j % _NBUF)


@jax.jit
def kernel(input_ids, token_table, position_table, ctx):
  ids = input_ids.astype(jnp.int32).reshape(-1)
  prefix = ctx[:N_CTX] + position_table[:N_CTX]
  base = jnp.concatenate([prefix, position_table[N_CTX:L]], axis=0)

  mesh = plsc.VectorSubcoreMesh(core_axis_name="c", subcore_axis_name="s")
  run = pl.kernel(
      _sc_embed,
      out_type=jax.ShapeDtypeStruct((B, L, EMBED_DIM), jnp.float32),
      mesh=mesh,
      scratch_types=[
          pltpu.VMEM((_BPW * L,), jnp.int32),
          pltpu.VMEM((_NBUF, L, EMBED_DIM), jnp.float32),
          pltpu.VMEM_SHARED((L, EMBED_DIM), jnp.float32),
          pltpu.SemaphoreType.DMA((4,)),
          pltpu.SemaphoreType.DMA((4,)),
          pltpu.SemaphoreType.DMA((4,)),
      ],
  )
  return run(ids, base, token_table)
